# baseline (device time: 850579 ns/iter reference)
import jax
import jax.numpy as jnp
from jax import lax
from jax.experimental import pallas as pl
from jax.experimental.pallas import tpu as pltpu

N_DEV = 4


def kernel(A, B):
    m_per, k = A.shape
    n = B.shape[1]
    half = m_per // 2

    def body(a32_ref, b32_ref, out_ref, far_ref, bbf_ref,
             a_ref, b_ref, cl_ref, cr_ref, cf_ref,
             stage_ref, send_sems, recv_sems, store_sems):
        my = lax.axis_index("i")
        left = lax.rem(my + N_DEV - 1, N_DEV)
        right = lax.rem(my + 1, N_DEV)

        a_ref[...] = a32_ref[...].astype(jnp.bfloat16)

        barrier_sem = pltpu.get_barrier_semaphore()
        pl.semaphore_signal(
            barrier_sem, inc=1, device_id=(left,),
            device_id_type=pl.DeviceIdType.MESH,
        )
        pl.semaphore_signal(
            barrier_sem, inc=1, device_id=(right,),
            device_id_type=pl.DeviceIdType.MESH,
        )
        pl.semaphore_wait(barrier_sem, 2)

        r1 = pltpu.make_async_remote_copy(
            src_ref=a_ref, dst_ref=cl_ref,
            send_sem=send_sems.at[0], recv_sem=recv_sems.at[0],
            device_id=(right,), device_id_type=pl.DeviceIdType.MESH,
        )
        l1 = pltpu.make_async_remote_copy(
            src_ref=a_ref, dst_ref=cr_ref,
            send_sem=send_sems.at[1], recv_sem=recv_sems.at[1],
            device_id=(left,), device_id_type=pl.DeviceIdType.MESH,
        )
        r2 = pltpu.make_async_remote_copy(
            src_ref=cl_ref.at[pl.ds(0, half), :],
            dst_ref=cf_ref.at[pl.ds(0, half), :],
            send_sem=send_sems.at[2], recv_sem=recv_sems.at[2],
            device_id=(right,), device_id_type=pl.DeviceIdType.MESH,
        )
        l2 = pltpu.make_async_remote_copy(
            src_ref=cr_ref.at[pl.ds(half, half), :],
            dst_ref=cf_ref.at[pl.ds(half, half), :],
            send_sem=send_sems.at[3], recv_sem=recv_sems.at[3],
            device_id=(left,), device_id_type=pl.DeviceIdType.MESH,
        )

        r1.start()
        l1.start()

        b_ref[...] = b32_ref[...].astype(jnp.bfloat16)
        b = b_ref[...]
        stb = pltpu.make_async_copy(b_ref, bbf_ref, store_sems.at[3])
        stb.start()

        def compute_store(src_ref, origin, slot):
            stage_ref[slot] = jnp.dot(
                src_ref[...], b, preferred_element_type=jnp.float32
            ).astype(jnp.bfloat16)
            st = pltpu.make_async_copy(
                stage_ref.at[slot],
                out_ref.at[pl.ds(origin * m_per, m_per), :],
                store_sems.at[slot],
            )
            st.start()
            return st

        st0 = compute_store(a_ref, my, 0)

        r1.wait_recv()
        r2.start()
        l1.wait_recv()
        l2.start()

        st1 = compute_store(cl_ref, left, 1)
        st0.wait()
        st2 = compute_store(cr_ref, right, 0)

        r2.wait_recv()
        l2.wait_recv()
        stf = pltpu.make_async_copy(cf_ref, far_ref, store_sems.at[2])
        stf.start()

        st1.wait()
        st2.wait()
        stf.wait()
        stb.wait()
        r1.wait_send()
        l1.wait_send()
        r2.wait_send()
        l2.wait_send()

    partial, far, bbf = pl.pallas_call(
        body,
        out_shape=(
            jax.ShapeDtypeStruct((N_DEV * m_per, n), jnp.bfloat16),
            jax.ShapeDtypeStruct((m_per, k), jnp.bfloat16),
            jax.ShapeDtypeStruct((k, n), jnp.bfloat16),
        ),
        in_specs=[
            pl.BlockSpec(memory_space=pltpu.MemorySpace.VMEM),
            pl.BlockSpec(memory_space=pltpu.MemorySpace.VMEM),
        ],
        out_specs=(
            pl.BlockSpec(memory_space=pl.ANY),
            pl.BlockSpec(memory_space=pl.ANY),
            pl.BlockSpec(memory_space=pl.ANY),
        ),
        scratch_shapes=[
            pltpu.VMEM((m_per, k), jnp.bfloat16),
            pltpu.VMEM((k, n), jnp.bfloat16),
            pltpu.VMEM((m_per, k), jnp.bfloat16),
            pltpu.VMEM((m_per, k), jnp.bfloat16),
            pltpu.VMEM((m_per, k), jnp.bfloat16),
            pltpu.VMEM((2, m_per, n), jnp.bfloat16),
            pltpu.SemaphoreType.DMA((4,)),
            pltpu.SemaphoreType.DMA((4,)),
            pltpu.SemaphoreType.DMA((4,)),
        ],
        compiler_params=pltpu.CompilerParams(
            collective_id=0, vmem_limit_bytes=110 * 1024 * 1024
        ),
    )(A, B)

    def body2(partial_ref, far_ref, b_ref, out_ref, stage_ref, sems):
        far_idx = lax.rem(lax.axis_index("i") + 2, N_DEV)

        copies = []
        for j in range(N_DEV):
            cp = pltpu.make_async_copy(
                partial_ref.at[pl.ds(j * m_per, m_per), :],
                out_ref.at[pl.ds(j * m_per, m_per), :],
                sems.at[j],
            )

            @pl.when(j != far_idx)
            def _(cp=cp):
                cp.start()

            copies.append(cp)

        stage_ref[pl.ds(0, half), :] = jnp.dot(
            far_ref[pl.ds(0, half), :], b_ref[...],
            preferred_element_type=jnp.float32,
        ).astype(jnp.bfloat16)
        stage_ref[pl.ds(half, half), :] = jnp.dot(
            far_ref[pl.ds(half, half), :], b_ref[...],
            preferred_element_type=jnp.float32,
        ).astype(jnp.bfloat16)
        stf = pltpu.make_async_copy(
            stage_ref, out_ref.at[pl.ds(far_idx * m_per, m_per), :],
            sems.at[N_DEV],
        )
        stf.start()
        stf.wait()

        for j, cp in enumerate(copies):
            @pl.when(j != far_idx)
            def _(cp=cp):
                cp.wait()

    return pl.pallas_call(
        body2,
        out_shape=jax.ShapeDtypeStruct((N_DEV * m_per, n), jnp.bfloat16),
        in_specs=[
            pl.BlockSpec(memory_space=pl.ANY),
            pl.BlockSpec(memory_space=pltpu.MemorySpace.VMEM),
            pl.BlockSpec(memory_space=pltpu.MemorySpace.VMEM),
        ],
        out_specs=pl.BlockSpec(memory_space=pl.ANY),
        scratch_shapes=[
            pltpu.VMEM((m_per, n), jnp.bfloat16),
            pltpu.SemaphoreType.DMA((N_DEV + 1,)),
        ],
        compiler_params=pltpu.CompilerParams(
            vmem_limit_bytes=110 * 1024 * 1024
        ),
    )(partial, far, bbf)


# device time: 112995 ns/iter; 7.5276x vs baseline; 7.5276x over previous
import jax
import jax.numpy as jnp
from jax import lax
from jax.experimental import pallas as pl
from jax.experimental.pallas import tpu as pltpu

N_DEV = 4


def kernel(A, B):
    m_per, k = A.shape
    n = B.shape[1]
    half = m_per // 2

    def body(a32_ref, b32_ref, out_ref, far_ref, bbf_ref,
             a_ref, b_ref, cl_ref, cr_ref, cf_ref,
             stage_ref, send_sems, recv_sems, store_sems):
        my = lax.axis_index("i")
        left = lax.rem(my + N_DEV - 1, N_DEV)
        right = lax.rem(my + 1, N_DEV)

        a_ref[...] = a32_ref[...].astype(jnp.bfloat16)

        barrier_sem = pltpu.get_barrier_semaphore()
        pl.semaphore_signal(
            barrier_sem, inc=1, device_id=(left,),
            device_id_type=pl.DeviceIdType.MESH,
        )
        pl.semaphore_signal(
            barrier_sem, inc=1, device_id=(right,),
            device_id_type=pl.DeviceIdType.MESH,
        )
        pl.semaphore_wait(barrier_sem, 2)

        r1 = pltpu.make_async_remote_copy(
            src_ref=a_ref, dst_ref=cl_ref,
            send_sem=send_sems.at[0], recv_sem=recv_sems.at[0],
            device_id=(right,), device_id_type=pl.DeviceIdType.MESH,
        )
        l1 = pltpu.make_async_remote_copy(
            src_ref=a_ref, dst_ref=cr_ref,
            send_sem=send_sems.at[1], recv_sem=recv_sems.at[1],
            device_id=(left,), device_id_type=pl.DeviceIdType.MESH,
        )
        r2 = pltpu.make_async_remote_copy(
            src_ref=cl_ref.at[pl.ds(0, half), :],
            dst_ref=cf_ref.at[pl.ds(0, half), :],
            send_sem=send_sems.at[2], recv_sem=recv_sems.at[2],
            device_id=(right,), device_id_type=pl.DeviceIdType.MESH,
        )
        l2 = pltpu.make_async_remote_copy(
            src_ref=cr_ref.at[pl.ds(half, half), :],
            dst_ref=cf_ref.at[pl.ds(half, half), :],
            send_sem=send_sems.at[3], recv_sem=recv_sems.at[3],
            device_id=(left,), device_id_type=pl.DeviceIdType.MESH,
        )

        r1.start()
        l1.start()

        b_ref[...] = b32_ref[...].astype(jnp.bfloat16)
        b = b_ref[...]
        stb = pltpu.make_async_copy(b_ref, bbf_ref, store_sems.at[3])
        stb.start()

        def compute_store(src_ref, origin, slot):
            stage_ref[slot] = jnp.dot(
                src_ref[...], b, preferred_element_type=jnp.float32
            ).astype(jnp.bfloat16)
            st = pltpu.make_async_copy(
                stage_ref.at[slot],
                out_ref.at[pl.ds(origin * m_per, m_per), :],
                store_sems.at[slot],
            )
            st.start()
            return st

        st0 = compute_store(a_ref, my, 0)

        r1.wait_recv()
        r2.start()
        l1.wait_recv()
        l2.start()

        st1 = compute_store(cl_ref, left, 1)
        st0.wait()
        st2 = compute_store(cr_ref, right, 0)

        r2.wait_recv()
        l2.wait_recv()
        stf = pltpu.make_async_copy(cf_ref, far_ref, store_sems.at[2])
        stf.start()

        st1.wait()
        st2.wait()
        stf.wait()
        stb.wait()
        r1.wait_send()
        l1.wait_send()
        r2.wait_send()
        l2.wait_send()

    partial, far, bbf = pl.pallas_call(
        body,
        out_shape=(
            jax.ShapeDtypeStruct((N_DEV * m_per, n), jnp.bfloat16),
            jax.ShapeDtypeStruct((m_per, k), jnp.bfloat16),
            jax.ShapeDtypeStruct((k, n), jnp.bfloat16),
        ),
        in_specs=[
            pl.BlockSpec(memory_space=pltpu.MemorySpace.VMEM),
            pl.BlockSpec(memory_space=pltpu.MemorySpace.VMEM),
        ],
        out_specs=(
            pl.BlockSpec(memory_space=pl.ANY),
            pl.BlockSpec(memory_space=pl.ANY),
            pl.BlockSpec(memory_space=pl.ANY),
        ),
        scratch_shapes=[
            pltpu.VMEM((m_per, k), jnp.bfloat16),
            pltpu.VMEM((k, n), jnp.bfloat16),
            pltpu.VMEM((m_per, k), jnp.bfloat16),
            pltpu.VMEM((m_per, k), jnp.bfloat16),
            pltpu.VMEM((m_per, k), jnp.bfloat16),
            pltpu.VMEM((2, m_per, n), jnp.bfloat16),
            pltpu.SemaphoreType.DMA((4,)),
            pltpu.SemaphoreType.DMA((4,)),
            pltpu.SemaphoreType.DMA((4,)),
        ],
        compiler_params=pltpu.CompilerParams(
            collective_id=0, vmem_limit_bytes=110 * 1024 * 1024
        ),
    )(A, B)

    n_blk = 2 * N_DEV

    def body2(partial_ref, far_ref, b_ref, out_ref):
        j = pl.program_id(0)
        far_idx = lax.rem(lax.axis_index("i") + 2, N_DEV)

        @pl.when(j == 2 * far_idx)
        def _():
            out_ref[...] = jnp.dot(
                far_ref[pl.ds(0, half), :], b_ref[...],
                preferred_element_type=jnp.float32,
            ).astype(jnp.bfloat16)

        @pl.when(j == 2 * far_idx + 1)
        def _():
            out_ref[...] = jnp.dot(
                far_ref[pl.ds(half, half), :], b_ref[...],
                preferred_element_type=jnp.float32,
            ).astype(jnp.bfloat16)

        @pl.when(jnp.logical_and(j != 2 * far_idx, j != 2 * far_idx + 1))
        def _():
            out_ref[...] = partial_ref[...]

    return pl.pallas_call(
        body2,
        grid=(n_blk,),
        out_shape=jax.ShapeDtypeStruct((N_DEV * m_per, n), jnp.bfloat16),
        in_specs=[
            pl.BlockSpec((half, n), lambda j: (j, 0)),
            pl.BlockSpec((m_per, k), lambda j: (0, 0)),
            pl.BlockSpec((k, n), lambda j: (0, 0)),
        ],
        out_specs=pl.BlockSpec((half, n), lambda j: (j, 0)),
        compiler_params=pltpu.CompilerParams(
            vmem_limit_bytes=110 * 1024 * 1024
        ),
    )(partial, far, bbf)
